# Initial kernel scaffold; baseline (speedup 1.0000x reference)
import jax
import jax.numpy as jnp
from jax.experimental import pallas as pl


def kernel(context_words, target_words, negative_words, W_in, W_out):
    raise NotImplementedError("write your pallas kernel here")


# speedup vs baseline: 1.1643x; 1.1643x over previous
import functools

import jax
import jax.numpy as jnp
from jax import lax
from jax.experimental import pallas as pl
from jax.experimental.pallas import tpu as pltpu
from jax.experimental.pallas import tpu_sc as plsc

_VOCAB = 1000000
_D = 32
_B = 16384
_CTX = 20
_NEG = 20

_NC = 2
_NS = 16
_NW = _NC * _NS
_L = 16

_BPW = _B // _NW
_C = 32
_NCHUNK = _BPW // _C
_GPC = _C // _L
_RPC = _C * _CTX
_IDXROWS = _RPC // 128


def _sc_body(ctx_idx_hbm, tgt_idx_hbm, neg_idx_hbm, win_hbm, wout_hbm,
             pos_out, neg_out,
             ctx_idx_v, neg_idx_v, tgt_idx_v,
             ctx_rows, neg_rows, tgt_rows,
             pos_acc, neg_acc, sem):
    wid = lax.axis_index("s") * _NC + lax.axis_index("c")
    iota = lax.iota(jnp.int32, _L)

    def chunk_body(c, _):
        base = wid * _BPW + c * _C
        flat0 = base * _CTX
        pltpu.sync_copy(ctx_idx_hbm.at[pl.ds(flat0, _RPC)], ctx_idx_v)
        pltpu.sync_copy(neg_idx_hbm.at[pl.ds(flat0, _RPC)], neg_idx_v)
        pltpu.sync_copy(tgt_idx_hbm.at[pl.ds(base, _C)], tgt_idx_v)
        handles = []
        for j in range(_IDXROWS):
            handles.append(pltpu.async_copy(
                win_hbm.at[ctx_idx_v.at[pl.ds(j * 128, 128)]],
                ctx_rows.at[pl.ds(j * 128, 128)], sem))
            handles.append(pltpu.async_copy(
                wout_hbm.at[neg_idx_v.at[pl.ds(j * 128, 128)]],
                neg_rows.at[pl.ds(j * 128, 128)], sem))
        handles.append(pltpu.async_copy(wout_hbm.at[tgt_idx_v], tgt_rows, sem))
        for h in handles:
            h.wait()

        def group_body(g, _):
            item0 = g * _L
            row0 = (item0 + iota) * _CTX
            trow = item0 + iota
            pos = jnp.zeros((_L,), jnp.float32)
            negs = [jnp.zeros((_L,), jnp.float32) for _ in range(_NEG)]
            for d in range(_D):
                col = jnp.full((_L,), d, jnp.int32)
                acc = plsc.load_gather(ctx_rows, [row0, col])
                for n in range(1, _CTX):
                    acc = acc + plsc.load_gather(ctx_rows, [row0 + n, col])
                cm = acc * (1.0 / _CTX)
                pos = pos + plsc.load_gather(tgt_rows, [trow, col]) * cm
                for n in range(_NEG):
                    negs[n] = negs[n] + plsc.load_gather(
                        neg_rows, [row0 + n, col]) * cm
            off = c * _C + item0
            pos_acc[pl.ds(off, _L)] = pos
            for n in range(_NEG):
                neg_acc[n, pl.ds(off, _L)] = negs[n]
            return 0

        lax.fori_loop(0, _GPC, group_body, 0)
        return 0

    lax.fori_loop(0, _NCHUNK, chunk_body, 0)
    pltpu.sync_copy(pos_acc, pos_out.at[pl.ds(wid * _BPW, _BPW)])
    pltpu.sync_copy(neg_acc, neg_out.at[wid])


def _sc_scores(ctx_idx, tgt_idx, neg_idx, w_in, w_out):
    mesh = plsc.VectorSubcoreMesh(core_axis_name="c", subcore_axis_name="s",
                                  num_cores=_NC, num_subcores=_NS)
    return pl.kernel(
        _sc_body,
        out_type=[
            jax.ShapeDtypeStruct((_B,), jnp.float32),
            jax.ShapeDtypeStruct((_NW, _NEG, _BPW), jnp.float32),
        ],
        mesh=mesh,
        compiler_params=pltpu.CompilerParams(use_tc_tiling_on_sc=False,
                                             needs_layout_passes=False),
        scratch_types=[
            pltpu.VMEM((_RPC,), jnp.int32),
            pltpu.VMEM((_RPC,), jnp.int32),
            pltpu.VMEM((_C,), jnp.int32),
            pltpu.VMEM((_RPC, _D), jnp.float32),
            pltpu.VMEM((_RPC, _D), jnp.float32),
            pltpu.VMEM((_C, _D), jnp.float32),
            pltpu.VMEM((_BPW,), jnp.float32),
            pltpu.VMEM((_NEG, _BPW), jnp.float32),
            pltpu.SemaphoreType.DMA,
        ],
    )(ctx_idx, tgt_idx, neg_idx, w_in, w_out)


def _tc_loss_body(pos_ref, neg_ref, out_ref):
    pos = pos_ref[...]
    neg = neg_ref[...]
    lsp = jnp.sum(jnp.log(jax.nn.sigmoid(pos) + 1e-10))
    lsn = jnp.sum(jnp.log(jax.nn.sigmoid(-neg) + 1e-10))
    out_ref[0, 0] = -(lsp + lsn) / _B


@jax.jit
def kernel(context_words, target_words, negative_words, W_in, W_out):
    ctx2d = context_words.astype(jnp.int32).reshape(_B * _CTX)
    neg2d = negative_words.astype(jnp.int32).reshape(_B * _NEG)
    tgt = target_words.astype(jnp.int32)
    pos_sc, neg_sc = _sc_scores(ctx2d, tgt, neg2d, W_in, W_out)
    loss = pl.pallas_call(
        _tc_loss_body,
        out_shape=jax.ShapeDtypeStruct((1, 1), jnp.float32),
        out_specs=pl.BlockSpec(memory_space=pltpu.SMEM),
    )(pos_sc.reshape(128, 128), neg_sc.reshape(_NW * _NEG, _BPW))
    return loss[0, 0]


# speedup vs baseline: 1.1679x; 1.0031x over previous
import functools

import jax
import jax.numpy as jnp
from jax import lax
from jax.experimental import pallas as pl
from jax.experimental.pallas import tpu as pltpu
from jax.experimental.pallas import tpu_sc as plsc

_VOCAB = 1000000
_D = 32
_B = 16384
_CTX = 20
_NEG = 20

_NC = 2
_NS = 16
_NW = _NC * _NS
_L = 16

_BPW = _B // _NW
_C = 32
_NCHUNK = _BPW // _C
_GPC = _C // _L
_RPC = _C * _CTX
_IDXROWS = _RPC // 128


def _sc_body(ctx_idx_hbm, tgt_idx_hbm, neg_idx_hbm, win_hbm, wout_hbm,
             pos_out, neg_out,
             ctx_idx2d, neg_idx2d, ctx_idx_v, neg_idx_v, tgt_idx_v,
             ctx_rows, neg_rows, tgt_rows,
             pos_acc, neg_acc, sem):
    wid = lax.axis_index("s") * _NC + lax.axis_index("c")
    iota = lax.iota(jnp.int32, _L)

    pltpu.sync_copy(ctx_idx_hbm.at[pl.ds(wid * _BPW, _BPW)], ctx_idx2d)
    pltpu.sync_copy(neg_idx_hbm.at[pl.ds(wid * _BPW, _BPW)], neg_idx2d)

    flat_items = [(j * _L + iota) // _CTX for j in range(_RPC // _L)]
    flat_slots = [(j * _L + iota) % _CTX for j in range(_RPC // _L)]

    def chunk_body(c, _):
        base = wid * _BPW + c * _C
        for j in range(_RPC // _L):
            items = flat_items[j] + c * _C
            ctx_idx_v[pl.ds(j * _L, _L)] = plsc.load_gather(
                ctx_idx2d, [items, flat_slots[j]])
            neg_idx_v[pl.ds(j * _L, _L)] = plsc.load_gather(
                neg_idx2d, [items, flat_slots[j]])
        pltpu.sync_copy(tgt_idx_hbm.at[pl.ds(base, _C)], tgt_idx_v)
        handles = []
        for j in range(_IDXROWS):
            handles.append(pltpu.async_copy(
                win_hbm.at[ctx_idx_v.at[pl.ds(j * 128, 128)]],
                ctx_rows.at[pl.ds(j * 128, 128)], sem))
            handles.append(pltpu.async_copy(
                wout_hbm.at[neg_idx_v.at[pl.ds(j * 128, 128)]],
                neg_rows.at[pl.ds(j * 128, 128)], sem))
        handles.append(pltpu.async_copy(wout_hbm.at[tgt_idx_v], tgt_rows, sem))
        for h in handles:
            h.wait()

        def group_body(g, _):
            item0 = g * _L
            row0 = (item0 + iota) * _CTX
            trow = item0 + iota
            pos = jnp.zeros((_L,), jnp.float32)
            negs = [jnp.zeros((_L,), jnp.float32) for _ in range(_NEG)]
            for d in range(_D):
                col = jnp.full((_L,), d, jnp.int32)
                acc = plsc.load_gather(ctx_rows, [row0, col])
                for n in range(1, _CTX):
                    acc = acc + plsc.load_gather(ctx_rows, [row0 + n, col])
                cm = acc * (1.0 / _CTX)
                pos = pos + plsc.load_gather(tgt_rows, [trow, col]) * cm
                for n in range(_NEG):
                    negs[n] = negs[n] + plsc.load_gather(
                        neg_rows, [row0 + n, col]) * cm
            off = c * _C + item0
            pos_acc[pl.ds(off, _L)] = pos
            for n in range(_NEG):
                neg_acc[n, pl.ds(off, _L)] = negs[n]
            return 0

        lax.fori_loop(0, _GPC, group_body, 0)
        return 0

    lax.fori_loop(0, _NCHUNK, chunk_body, 0)
    pltpu.sync_copy(pos_acc, pos_out.at[pl.ds(wid * _BPW, _BPW)])
    pltpu.sync_copy(neg_acc, neg_out.at[wid])


def _sc_scores(ctx_idx, tgt_idx, neg_idx, w_in, w_out):
    mesh = plsc.VectorSubcoreMesh(core_axis_name="c", subcore_axis_name="s",
                                  num_cores=_NC, num_subcores=_NS)
    return pl.kernel(
        _sc_body,
        out_type=[
            jax.ShapeDtypeStruct((_B,), jnp.float32),
            jax.ShapeDtypeStruct((_NW, _NEG, _BPW), jnp.float32),
        ],
        mesh=mesh,
        compiler_params=pltpu.CompilerParams(use_tc_tiling_on_sc=False,
                                             needs_layout_passes=False),
        scratch_types=[
            pltpu.VMEM((_BPW, _CTX), jnp.int32),
            pltpu.VMEM((_BPW, _NEG), jnp.int32),
            pltpu.VMEM((_RPC,), jnp.int32),
            pltpu.VMEM((_RPC,), jnp.int32),
            pltpu.VMEM((_C,), jnp.int32),
            pltpu.VMEM((_RPC, _D), jnp.float32),
            pltpu.VMEM((_RPC, _D), jnp.float32),
            pltpu.VMEM((_C, _D), jnp.float32),
            pltpu.VMEM((_BPW,), jnp.float32),
            pltpu.VMEM((_NEG, _BPW), jnp.float32),
            pltpu.SemaphoreType.DMA,
        ],
    )(ctx_idx, tgt_idx, neg_idx, w_in, w_out)


def _tc_loss_body(pos_ref, neg_ref, out_ref):
    pos = pos_ref[...]
    neg = neg_ref[...]
    lsp = jnp.sum(jnp.log(jax.nn.sigmoid(pos) + 1e-10))
    lsn = jnp.sum(jnp.log(jax.nn.sigmoid(-neg) + 1e-10))
    out_ref[0, 0] = -(lsp + lsn) / _B


@jax.jit
def kernel(context_words, target_words, negative_words, W_in, W_out):
    ctx2d = context_words.astype(jnp.int32)
    neg2d = negative_words.astype(jnp.int32)
    tgt = target_words.astype(jnp.int32)
    pos_sc, neg_sc = _sc_scores(ctx2d, tgt, neg2d, W_in, W_out)
    loss = pl.pallas_call(
        _tc_loss_body,
        out_shape=jax.ShapeDtypeStruct((1, 1), jnp.float32),
        out_specs=pl.BlockSpec(memory_space=pltpu.SMEM),
    )(pos_sc.reshape(128, 128), neg_sc.reshape(_NW * _NEG, _BPW))
    return loss[0, 0]
